# baseline (speedup 1.0000x reference)
import functools

import jax
import jax.numpy as jnp
from jax import lax
from jax.experimental import pallas as pl
from jax.experimental.pallas import tpu as pltpu
from jax.experimental.pallas import tpu_sc as plsc

V = 100000
H = 128
B = 1024
S = 50
N = B * S

_NC = 2
_NS = 16
NW = _NC * _NS
ROWS_PER_W = N // NW
CB = 80
NCHUNK = ROWS_PER_W // CB


def _sc_gather(emb, idx3):
    mesh = plsc.VectorSubcoreMesh(core_axis_name="c", subcore_axis_name="s")

    @functools.partial(
        pl.kernel,
        out_type=jax.ShapeDtypeStruct((NW, NCHUNK, CB, H), jnp.float32),
        mesh=mesh,
        scratch_types=[
            pltpu.VMEM((NCHUNK, CB), jnp.int32),
            pltpu.VMEM((2, CB, H), jnp.float32),
            pltpu.SemaphoreType.DMA,
            pltpu.SemaphoreType.DMA,
            pltpu.SemaphoreType.DMA,
            pltpu.SemaphoreType.DMA,
        ],
    )
    def k(emb_hbm, idx_hbm, out_hbm, idx_v, rows_v, g0, g1, o0, o1):
        wid = lax.axis_index("s") * _NC + lax.axis_index("c")
        pltpu.sync_copy(idx_hbm.at[wid], idx_v)
        gsems = (g0, g1)
        osems = (o0, o1)

        gathers = [None, None]
        outs = [None, None]
        gathers[0] = pltpu.async_copy(emb_hbm.at[idx_v.at[0]], rows_v.at[0], g0)
        for j in range(NCHUNK):
            b = j % 2
            nb = (j + 1) % 2
            if j + 1 < NCHUNK:
                if outs[nb] is not None:
                    outs[nb].wait()
                    outs[nb] = None
                gathers[nb] = pltpu.async_copy(
                    emb_hbm.at[idx_v.at[j + 1]], rows_v.at[nb], gsems[nb])
            gathers[b].wait()
            outs[b] = pltpu.async_copy(rows_v.at[b], out_hbm.at[wid, j], osems[b])
        outs[(NCHUNK - 1) % 2].wait()
        if outs[NCHUNK % 2] is not None:
            outs[NCHUNK % 2].wait()

    return k(emb, idx3)


def _gru_body(x_ref, wih_ref, whh_ref, bih_ref, bhh_ref, out_ref, h_ref):
    t = pl.program_id(0)

    @pl.when(t == 0)
    def _():
        h_ref[...] = jnp.zeros_like(h_ref)

    x = x_ref[0]
    h = h_ref[...]
    gi = jnp.dot(x, wih_ref[...], preferred_element_type=jnp.float32) + bih_ref[...]
    gh = jnp.dot(h, whh_ref[...], preferred_element_type=jnp.float32) + bhh_ref[...]
    r = 0.5 + 0.5 * jnp.tanh(0.5 * (gi[:, :H] + gh[:, :H]))
    z = 0.5 + 0.5 * jnp.tanh(0.5 * (gi[:, H:2 * H] + gh[:, H:2 * H]))
    n = jnp.tanh(gi[:, 2 * H:] + r * gh[:, 2 * H:])
    h_new = n + z * (h - n)
    h_ref[...] = h_new
    out_ref[0] = h_new


def _gru(x_sbh, wih_t, whh_t, bih2, bhh2):
    return pl.pallas_call(
        _gru_body,
        grid=(S,),
        in_specs=[
            pl.BlockSpec((1, B, H), lambda t: (t, 0, 0)),
            pl.BlockSpec((H, 3 * H), lambda t: (0, 0)),
            pl.BlockSpec((H, 3 * H), lambda t: (0, 0)),
            pl.BlockSpec((1, 3 * H), lambda t: (0, 0)),
            pl.BlockSpec((1, 3 * H), lambda t: (0, 0)),
        ],
        out_specs=pl.BlockSpec((1, B, H), lambda t: (t, 0, 0)),
        out_shape=jax.ShapeDtypeStruct((S, B, H), jnp.float32),
        scratch_shapes=[pltpu.VMEM((B, H), jnp.float32)],
        compiler_params=pltpu.CompilerParams(
            dimension_semantics=("arbitrary",)),
    )(x_sbh, wih_t, whh_t, bih2, bhh2)


def kernel(input, emb, W_ih, W_hh, b_ih, b_hh):
    idx3 = input.astype(jnp.int32).T.reshape(NW, NCHUNK, CB)
    x = _sc_gather(emb, idx3)
    x_sbh = x.reshape(S, B, H)
    enc_t = _gru(x_sbh, W_ih.T, W_hh.T,
                 b_ih.reshape(1, 3 * H), b_hh.reshape(1, 3 * H))
    enc = jnp.transpose(enc_t, (1, 0, 2))
    hidden = enc_t[S - 1][None]
    return (enc, hidden)


# speedup vs baseline: 1.1670x; 1.1670x over previous
import functools

import jax
import jax.numpy as jnp
from jax import lax
from jax.experimental import pallas as pl
from jax.experimental.pallas import tpu as pltpu
from jax.experimental.pallas import tpu_sc as plsc

V = 100000
H = 128
B = 1024
S = 50
N = B * S

_NC = 2
_NS = 16
NW = _NC * _NS
ROWS_PER_W = N // NW
CB = 80
NCHUNK = ROWS_PER_W // CB


def _sc_gather(emb, idx3):
    mesh = plsc.VectorSubcoreMesh(core_axis_name="c", subcore_axis_name="s")

    @functools.partial(
        pl.kernel,
        out_type=jax.ShapeDtypeStruct((NW, NCHUNK, CB, H), jnp.float32),
        mesh=mesh,
        scratch_types=[
            pltpu.VMEM((NCHUNK, CB), jnp.int32),
            pltpu.VMEM((2, CB, H), jnp.float32),
            pltpu.SemaphoreType.DMA,
            pltpu.SemaphoreType.DMA,
            pltpu.SemaphoreType.DMA,
            pltpu.SemaphoreType.DMA,
        ],
    )
    def k(emb_hbm, idx_hbm, out_hbm, idx_v, rows_v, g0, g1, o0, o1):
        wid = lax.axis_index("s") * _NC + lax.axis_index("c")
        pltpu.sync_copy(idx_hbm.at[wid], idx_v)
        gsems = (g0, g1)
        osems = (o0, o1)

        gathers = [None, None]
        outs = [None, None]
        gathers[0] = pltpu.async_copy(emb_hbm.at[idx_v.at[0]], rows_v.at[0], g0)
        for j in range(NCHUNK):
            b = j % 2
            nb = (j + 1) % 2
            if j + 1 < NCHUNK:
                if outs[nb] is not None:
                    outs[nb].wait()
                    outs[nb] = None
                gathers[nb] = pltpu.async_copy(
                    emb_hbm.at[idx_v.at[j + 1]], rows_v.at[nb], gsems[nb])
            gathers[b].wait()
            outs[b] = pltpu.async_copy(rows_v.at[b], out_hbm.at[wid, j], osems[b])
        outs[(NCHUNK - 1) % 2].wait()
        if outs[NCHUNK % 2] is not None:
            outs[NCHUNK % 2].wait()

    return k(emb, idx3)


TSTEPS = 2


def _gru_body(x_ref, wih_ref, whh_ref, bih_ref, bhh_ref, out_ref, h_ref):
    g = pl.program_id(0)

    @pl.when(g == 0)
    def _():
        h_ref[...] = jnp.zeros_like(h_ref)

    h = h_ref[...]
    wih = wih_ref[...]
    whh = whh_ref[...]
    for k in range(TSTEPS):
        x = x_ref[k]
        gi = jnp.dot(x, wih, preferred_element_type=jnp.float32) + bih_ref[...]
        gh = jnp.dot(h, whh, preferred_element_type=jnp.float32) + bhh_ref[...]
        r = 0.5 + 0.5 * jnp.tanh(0.5 * (gi[:, :H] + gh[:, :H]))
        z = 0.5 + 0.5 * jnp.tanh(0.5 * (gi[:, H:2 * H] + gh[:, H:2 * H]))
        n = jnp.tanh(gi[:, 2 * H:] + r * gh[:, 2 * H:])
        h = n + z * (h - n)
        out_ref[k] = h
    h_ref[...] = h


def _gru(x_sbh, wih_t, whh_t, bih2, bhh2):
    return pl.pallas_call(
        _gru_body,
        grid=(S // TSTEPS,),
        in_specs=[
            pl.BlockSpec((TSTEPS, B, H), lambda t: (t, 0, 0)),
            pl.BlockSpec((H, 3 * H), lambda t: (0, 0)),
            pl.BlockSpec((H, 3 * H), lambda t: (0, 0)),
            pl.BlockSpec((1, 3 * H), lambda t: (0, 0)),
            pl.BlockSpec((1, 3 * H), lambda t: (0, 0)),
        ],
        out_specs=pl.BlockSpec((TSTEPS, B, H), lambda t: (t, 0, 0)),
        out_shape=jax.ShapeDtypeStruct((S, B, H), jnp.float32),
        scratch_shapes=[pltpu.VMEM((B, H), jnp.float32)],
        compiler_params=pltpu.CompilerParams(
            dimension_semantics=("arbitrary",)),
    )(x_sbh, wih_t, whh_t, bih2, bhh2)


def kernel(input, emb, W_ih, W_hh, b_ih, b_hh):
    idx3 = input.astype(jnp.int32).T.reshape(NW, NCHUNK, CB)
    x = _sc_gather(emb, idx3)
    x_sbh = x.reshape(S, B, H)
    enc_t = _gru(x_sbh, W_ih.T, W_hh.T,
                 b_ih.reshape(1, 3 * H), b_hh.reshape(1, 3 * H))
    enc = jnp.transpose(enc_t, (1, 0, 2))
    hidden = enc_t[S - 1][None]
    return (enc, hidden)


# speedup vs baseline: 1.2940x; 1.1088x over previous
import functools

import jax
import jax.numpy as jnp
from jax import lax
from jax.experimental import pallas as pl
from jax.experimental.pallas import tpu as pltpu
from jax.experimental.pallas import tpu_sc as plsc

V = 100000
H = 128
B = 1024
S = 50
N = B * S

_NC = 2
_NS = 16
NW = _NC * _NS
ROWS_PER_W = N // NW
CB = 80
NCHUNK = ROWS_PER_W // CB


def _sc_gather(emb, idx3):
    mesh = plsc.VectorSubcoreMesh(core_axis_name="c", subcore_axis_name="s")

    @functools.partial(
        pl.kernel,
        out_type=jax.ShapeDtypeStruct((NW, NCHUNK, CB, H), jnp.float32),
        mesh=mesh,
        scratch_types=[
            pltpu.VMEM((NCHUNK, CB), jnp.int32),
            pltpu.VMEM((2, CB, H), jnp.float32),
            pltpu.SemaphoreType.DMA,
            pltpu.SemaphoreType.DMA,
            pltpu.SemaphoreType.DMA,
            pltpu.SemaphoreType.DMA,
        ],
    )
    def k(emb_hbm, idx_hbm, out_hbm, idx_v, rows_v, g0, g1, o0, o1):
        wid = lax.axis_index("s") * _NC + lax.axis_index("c")
        pltpu.sync_copy(idx_hbm.at[wid], idx_v)
        gsems = (g0, g1)
        osems = (o0, o1)

        gathers = [None, None]
        outs = [None, None]
        gathers[0] = pltpu.async_copy(emb_hbm.at[idx_v.at[0]], rows_v.at[0], g0)
        for j in range(NCHUNK):
            b = j % 2
            nb = (j + 1) % 2
            if j + 1 < NCHUNK:
                if outs[nb] is not None:
                    outs[nb].wait()
                    outs[nb] = None
                gathers[nb] = pltpu.async_copy(
                    emb_hbm.at[idx_v.at[j + 1]], rows_v.at[nb], gsems[nb])
            gathers[b].wait()
            outs[b] = pltpu.async_copy(rows_v.at[b], out_hbm.at[wid, j], osems[b])
        outs[(NCHUNK - 1) % 2].wait()
        if outs[NCHUNK % 2] is not None:
            outs[NCHUNK % 2].wait()

    return k(emb, idx3)


TSTEPS = 5


def _gru_body(x_ref, wih_ref, whh_ref, bih_ref, bhh_ref, out_ref, h_ref):
    g = pl.program_id(0)

    @pl.when(g == 0)
    def _():
        h_ref[...] = jnp.zeros_like(h_ref)

    h = h_ref[...]
    wih = wih_ref[...]
    whh = whh_ref[...]
    for k in range(TSTEPS):
        x = x_ref[k]
        gi = jnp.dot(x, wih, preferred_element_type=jnp.float32) + bih_ref[...]
        gh = jnp.dot(h, whh, preferred_element_type=jnp.float32) + bhh_ref[...]
        r = 0.5 + 0.5 * jnp.tanh(0.5 * (gi[:, :H] + gh[:, :H]))
        z = 0.5 + 0.5 * jnp.tanh(0.5 * (gi[:, H:2 * H] + gh[:, H:2 * H]))
        n = jnp.tanh(gi[:, 2 * H:] + r * gh[:, 2 * H:])
        h = n + z * (h - n)
        out_ref[k] = h
    h_ref[...] = h


def _gru(x_sbh, wih_t, whh_t, bih2, bhh2):
    return pl.pallas_call(
        _gru_body,
        grid=(S // TSTEPS,),
        in_specs=[
            pl.BlockSpec((TSTEPS, B, H), lambda t: (t, 0, 0)),
            pl.BlockSpec((H, 3 * H), lambda t: (0, 0)),
            pl.BlockSpec((H, 3 * H), lambda t: (0, 0)),
            pl.BlockSpec((1, 3 * H), lambda t: (0, 0)),
            pl.BlockSpec((1, 3 * H), lambda t: (0, 0)),
        ],
        out_specs=pl.BlockSpec((TSTEPS, B, H), lambda t: (t, 0, 0)),
        out_shape=jax.ShapeDtypeStruct((S, B, H), jnp.float32),
        scratch_shapes=[pltpu.VMEM((B, H), jnp.float32)],
        compiler_params=pltpu.CompilerParams(
            dimension_semantics=("arbitrary",)),
    )(x_sbh, wih_t, whh_t, bih2, bhh2)


def kernel(input, emb, W_ih, W_hh, b_ih, b_hh):
    idx3 = input.astype(jnp.int32).T.reshape(NW, NCHUNK, CB)
    x = _sc_gather(emb, idx3)
    x_sbh = x.reshape(S, B, H)
    enc_t = _gru(x_sbh, W_ih.T, W_hh.T,
                 b_ih.reshape(1, 3 * H), b_hh.reshape(1, 3 * H))
    enc = jnp.transpose(enc_t, (1, 0, 2))
    hidden = enc_t[S - 1][None]
    return (enc, hidden)


# speedup vs baseline: 1.3064x; 1.0096x over previous
import functools

import jax
import jax.numpy as jnp
from jax import lax
from jax.experimental import pallas as pl
from jax.experimental.pallas import tpu as pltpu
from jax.experimental.pallas import tpu_sc as plsc

V = 100000
H = 128
B = 1024
S = 50
N = B * S

_NC = 2
_NS = 16
NW = _NC * _NS
ROWS_PER_W = N // NW
CB = 80
NCHUNK = ROWS_PER_W // CB


def _sc_gather(emb, idx3):
    mesh = plsc.VectorSubcoreMesh(core_axis_name="c", subcore_axis_name="s")

    @functools.partial(
        pl.kernel,
        out_type=jax.ShapeDtypeStruct((NW, NCHUNK, CB, H), jnp.float32),
        mesh=mesh,
        scratch_types=[
            pltpu.VMEM((NCHUNK, CB), jnp.int32),
            pltpu.VMEM((2, CB, H), jnp.float32),
            pltpu.SemaphoreType.DMA,
            pltpu.SemaphoreType.DMA,
            pltpu.SemaphoreType.DMA,
            pltpu.SemaphoreType.DMA,
        ],
    )
    def k(emb_hbm, idx_hbm, out_hbm, idx_v, rows_v, g0, g1, o0, o1):
        wid = lax.axis_index("s") * _NC + lax.axis_index("c")
        pltpu.sync_copy(idx_hbm.at[wid], idx_v)
        gsems = (g0, g1)
        osems = (o0, o1)

        gathers = [None, None]
        outs = [None, None]
        gathers[0] = pltpu.async_copy(emb_hbm.at[idx_v.at[0]], rows_v.at[0], g0)
        for j in range(NCHUNK):
            b = j % 2
            nb = (j + 1) % 2
            if j + 1 < NCHUNK:
                if outs[nb] is not None:
                    outs[nb].wait()
                    outs[nb] = None
                gathers[nb] = pltpu.async_copy(
                    emb_hbm.at[idx_v.at[j + 1]], rows_v.at[nb], gsems[nb])
            gathers[b].wait()
            outs[b] = pltpu.async_copy(rows_v.at[b], out_hbm.at[wid, j], osems[b])
        outs[(NCHUNK - 1) % 2].wait()
        if outs[NCHUNK % 2] is not None:
            outs[NCHUNK % 2].wait()

    return k(emb, idx3)


TSTEPS = 10


def _gru_body(x_ref, wih_ref, whh_ref, bih_ref, bhh_ref, out_ref, h_ref):
    g = pl.program_id(0)

    @pl.when(g == 0)
    def _():
        h_ref[...] = jnp.zeros_like(h_ref)

    h = h_ref[...]
    wih = wih_ref[...]
    whh = whh_ref[...]
    for k in range(TSTEPS):
        x = x_ref[k]
        gi = jnp.dot(x, wih, preferred_element_type=jnp.float32) + bih_ref[...]
        gh = jnp.dot(h, whh, preferred_element_type=jnp.float32) + bhh_ref[...]
        r = 0.5 + 0.5 * jnp.tanh(0.5 * (gi[:, :H] + gh[:, :H]))
        z = 0.5 + 0.5 * jnp.tanh(0.5 * (gi[:, H:2 * H] + gh[:, H:2 * H]))
        n = jnp.tanh(gi[:, 2 * H:] + r * gh[:, 2 * H:])
        h = n + z * (h - n)
        out_ref[k] = h
    h_ref[...] = h


def _gru(x_sbh, wih_t, whh_t, bih2, bhh2):
    return pl.pallas_call(
        _gru_body,
        grid=(S // TSTEPS,),
        in_specs=[
            pl.BlockSpec((TSTEPS, B, H), lambda t: (t, 0, 0)),
            pl.BlockSpec((H, 3 * H), lambda t: (0, 0)),
            pl.BlockSpec((H, 3 * H), lambda t: (0, 0)),
            pl.BlockSpec((1, 3 * H), lambda t: (0, 0)),
            pl.BlockSpec((1, 3 * H), lambda t: (0, 0)),
        ],
        out_specs=pl.BlockSpec((TSTEPS, B, H), lambda t: (t, 0, 0)),
        out_shape=jax.ShapeDtypeStruct((S, B, H), jnp.float32),
        scratch_shapes=[pltpu.VMEM((B, H), jnp.float32)],
        compiler_params=pltpu.CompilerParams(
            dimension_semantics=("arbitrary",)),
    )(x_sbh, wih_t, whh_t, bih2, bhh2)


def kernel(input, emb, W_ih, W_hh, b_ih, b_hh):
    idx3 = input.astype(jnp.int32).T.reshape(NW, NCHUNK, CB)
    x = _sc_gather(emb, idx3)
    x_sbh = x.reshape(S, B, H)
    enc_t = _gru(x_sbh, W_ih.T, W_hh.T,
                 b_ih.reshape(1, 3 * H), b_hh.reshape(1, 3 * H))
    enc = jnp.transpose(enc_t, (1, 0, 2))
    hidden = enc_t[S - 1][None]
    return (enc, hidden)
